# baseline (speedup 1.0000x reference)
import functools

import jax
import jax.numpy as jnp
from jax import lax
from jax.experimental import pallas as pl
from jax.experimental.pallas import tpu as pltpu
from jax.experimental.pallas import tpu_sc as plsc

N = 10000
E = 160000
IN_DIM = 256
HID_DIM = 512
OUT_DIM = 512

NC = 2
NS = 16
N_PAD = 10240
ZROWS = N_PAD // NS
E_PAD = 163840
EPS = E_PAD // NS
BLK = 128
NBLK = EPS // BLK
C = 128

_f32 = jnp.float32

def _sc_mesh():
    return plsc.VectorSubcoreMesh(
        core_axis_name="c", subcore_axis_name="s", num_cores=NC, num_subcores=NS)


HALF = NBLK // 2


def _half_pass(table_h, acc, src_slab, dst_slab, sts, gsems, ssems):
    pltpu.async_copy(table_h.at[src_slab.at[0]], sts[0], gsems[0])

    @pl.loop(0, HALF, step=2)
    def _(b):
        for cur in range(2):
            bb = b + cur
            nxt = 1 - cur
            pltpu.make_async_copy(table_h.at[src_slab.at[bb]], sts[cur], gsems[cur]).wait()
            pltpu.async_copy(sts[cur], acc.at[dst_slab.at[bb]], ssems[cur], add=True)

            @pl.when(bb + 1 < HALF)
            def _(bb=bb, cur=cur, nxt=nxt):
                @pl.when(bb >= 1)
                def _():
                    pltpu.make_async_copy(sts[nxt], acc.at[dst_slab.at[0]], ssems[nxt]).wait()

                pltpu.async_copy(table_h.at[src_slab.at[bb + 1]], sts[nxt], gsems[nxt])

    for i in range(2):
        pltpu.make_async_copy(sts[i], acc.at[dst_slab.at[0]], ssems[i]).wait()


def _agg_pass(table_h, src2_h, dst2_h, acc, src_slab, dst_slab, sts, gsems, ssems, s):
    for h in range(2):
        row0 = s * NBLK + h * HALF
        pltpu.sync_copy(src2_h.at[pl.ds(row0, HALF)], src_slab)
        pltpu.sync_copy(dst2_h.at[pl.ds(row0, HALF)], dst_slab)
        _half_pass(table_h, acc, src_slab, dst_slab, sts, gsems, ssems)


DEG_NBLK = (E_PAD // 2) // NS // BLK
DEG_ROWS = E_PAD // BLK // 2


def _sc_agg_deg(x_lo, x_hi, src_p, dst_p, zblk, onesblk):
    @functools.partial(
        pl.kernel,
        out_type=tuple(jax.ShapeDtypeStruct((N_PAD, C), _f32) for _ in range(4)),
        mesh=_sc_mesh(),
        scratch_types=[
            pltpu.VMEM_SHARED((N_PAD, C), _f32),
            pltpu.VMEM((HALF, BLK), jnp.int32),
            pltpu.VMEM((HALF, BLK), jnp.int32),
            pltpu.VMEM((BLK, C), _f32),
            pltpu.VMEM((BLK, C), _f32),
            pltpu.SemaphoreType.DMA,
            pltpu.SemaphoreType.DMA,
            pltpu.SemaphoreType.DMA,
            pltpu.SemaphoreType.DMA,
        ],
    )
    def k(xlo_h, xhi_h, src2_h, dst2_h, zb_h, on_h,
          alo_h, ahi_h, dp0_h, dp1_h, acc, src_slab, dst_slab, st0, st1, g0, g1, s0, s1):
        c = lax.axis_index("c")
        s = lax.axis_index("s")
        rows = pl.ds(s * ZROWS, ZROWS)
        pltpu.sync_copy(zb_h, acc.at[rows])
        plsc.subcore_barrier()

        @pl.when(c == 0)
        def _():
            _agg_pass(xlo_h, src2_h, dst2_h, acc, src_slab, dst_slab, (st0, st1), (g0, g1), (s0, s1), s)

        @pl.when(c == 1)
        def _():
            _agg_pass(xhi_h, src2_h, dst2_h, acc, src_slab, dst_slab, (st0, st1), (g0, g1), (s0, s1), s)

        plsc.subcore_barrier()

        @pl.when(c == 0)
        def _():
            pltpu.sync_copy(acc.at[rows], alo_h.at[rows])

        @pl.when(c == 1)
        def _():
            pltpu.sync_copy(acc.at[rows], ahi_h.at[rows])

        pltpu.sync_copy(zb_h, acc.at[rows])
        pltpu.sync_copy(on_h, st0)
        pltpu.sync_copy(dst2_h.at[pl.ds(c * DEG_ROWS + s * DEG_NBLK, DEG_NBLK)], dst_slab)
        plsc.subcore_barrier()

        @pl.loop(0, DEG_NBLK, step=8)
        def _(b):
            for j in range(8):
                pltpu.async_copy(st0, acc.at[dst_slab.at[b + j]], g1, add=True)
            for j in range(8):
                pltpu.make_async_copy(st0, acc.at[dst_slab.at[b + j]], g1).wait()

        plsc.subcore_barrier()

        @pl.when(c == 0)
        def _():
            pltpu.sync_copy(acc.at[rows], dp0_h.at[rows])

        @pl.when(c == 1)
        def _():
            pltpu.sync_copy(acc.at[rows], dp1_h.at[rows])

    return k(x_lo, x_hi, src_p, dst_p, zblk, onesblk)


def _sc_agg4(h0, h1, h2, h3, src_p, dst_p, zblk):
    @functools.partial(
        pl.kernel,
        out_type=tuple(jax.ShapeDtypeStruct((N_PAD, C), _f32) for _ in range(4)),
        mesh=_sc_mesh(),
        scratch_types=[
            pltpu.VMEM_SHARED((N_PAD, C), _f32),
            pltpu.VMEM((HALF, BLK), jnp.int32),
            pltpu.VMEM((HALF, BLK), jnp.int32),
            pltpu.VMEM((BLK, C), _f32),
            pltpu.VMEM((BLK, C), _f32),
            pltpu.SemaphoreType.DMA,
            pltpu.SemaphoreType.DMA,
            pltpu.SemaphoreType.DMA,
            pltpu.SemaphoreType.DMA,
        ],
    )
    def k(h0_h, h1_h, h2_h, h3_h, src2_h, dst2_h, zb_h,
          o0_h, o1_h, o2_h, o3_h, acc, src_slab, dst_slab, st0, st1, g0, g1, s0, s1):
        c = lax.axis_index("c")
        s = lax.axis_index("s")
        rows = pl.ds(s * ZROWS, ZROWS)
        tables = (h0_h, h1_h, h2_h, h3_h)
        outs = (o0_h, o1_h, o2_h, o3_h)
        for p in range(2):
            pltpu.sync_copy(zb_h, acc.at[rows])
            plsc.subcore_barrier()
            for cc in range(NC):
                @pl.when(c == cc)
                def _(cc=cc, p=p):
                    _agg_pass(tables[2 * cc + p], src2_h, dst2_h, acc,
                              src_slab, dst_slab, (st0, st1), (g0, g1), (s0, s1), s)
            plsc.subcore_barrier()
            for cc in range(NC):
                @pl.when(c == cc)
                def _(cc=cc, p=p):
                    pltpu.sync_copy(acc.at[rows], outs[2 * cc + p].at[rows])
            if p == 0:
                plsc.subcore_barrier()

    return k(h0, h1, h2, h3, src_p, dst_p, zblk)


def _tc_mm(xs, ws, bias=None, degs=None, resid=None, relu=False, out_chunks=1):
    BN = 1000
    M = ws[0].shape[1]
    nx = len(xs)
    args = list(xs) + list(ws)
    in_specs = [pl.BlockSpec((BN, x.shape[1]), lambda i: (i, 0)) for x in xs]
    in_specs += [pl.BlockSpec(w.shape, lambda i: (0, 0)) for w in ws]
    have_bias, have_deg, have_resid = (bias is not None), (degs is not None), (resid is not None)
    if have_bias:
        in_specs.append(pl.BlockSpec((1, M), lambda i: (0, 0)))
        args.append(bias)
    if have_deg:
        for dp in degs:
            in_specs.append(pl.BlockSpec((BN, C), lambda i: (i, 0)))
            args.append(dp)
    if have_resid:
        in_specs.append(pl.BlockSpec((BN, M), lambda i: (i, 0)))
        args.append(resid)
    if out_chunks == 1:
        out_shape = jax.ShapeDtypeStruct((N, M), _f32)
        out_specs = pl.BlockSpec((BN, M), lambda i: (i, 0))
    else:
        Mc = M // out_chunks
        out_shape = tuple(jax.ShapeDtypeStruct((N, Mc), _f32) for _ in range(out_chunks))
        out_specs = tuple(pl.BlockSpec((BN, Mc), lambda i: (i, 0)) for _ in range(out_chunks))

    def body(*refs):
        xrs = refs[:nx]
        wrs = refs[nx:2 * nx]
        pos = 2 * nx
        acc = jnp.dot(xrs[0][...], wrs[0][...], preferred_element_type=_f32)
        for j in range(1, nx):
            acc = acc + jnp.dot(xrs[j][...], wrs[j][...], preferred_element_type=_f32)
        if have_bias:
            b_ref = refs[pos]; pos += 1
        if have_deg:
            d = refs[pos][:, 0:1] + refs[pos + 1][:, 0:1]; pos += 2
            acc = acc / jnp.maximum(d, 1.0)
        if have_bias:
            acc = acc + b_ref[...]
        if have_resid:
            acc = acc + refs[pos][...]; pos += 1
        if relu:
            acc = jnp.maximum(acc, 0.0)
        outs = refs[pos:]
        if out_chunks == 1:
            outs[0][...] = acc
        else:
            for q in range(out_chunks):
                outs[q][...] = acc[:, q * (M // out_chunks):(q + 1) * (M // out_chunks)]

    return pl.pallas_call(
        body, grid=(N // BN,), in_specs=in_specs,
        out_specs=out_specs, out_shape=out_shape)(*args)


def kernel(x, edge_index, W1_l, b1_l, W1_r, W2_l, b2_l, W2_r):
    src = edge_index[0].astype(jnp.int32)
    dst = edge_index[1].astype(jnp.int32)
    pad = E_PAD - E
    src_p = jnp.concatenate([src, jnp.zeros((pad,), jnp.int32)]).reshape(E_PAD // BLK, BLK)
    dst_p = jnp.concatenate([dst, jnp.full((pad,), N, jnp.int32)]).reshape(E_PAD // BLK, BLK)
    x_lo = x[:, :C]
    x_hi = x[:, C:]
    zblk = jnp.zeros((ZROWS, C), _f32)
    onesblk = jnp.ones((BLK, C), _f32)
    Wt1l = [W1_l[:, j * C:(j + 1) * C].T for j in range(IN_DIM // C)]
    Wt1r = W1_r.T
    Wt2l = [W2_l[:, j * C:(j + 1) * C].T for j in range(HID_DIM // C)]
    Wt2r = [W2_r[:, j * C:(j + 1) * C].T for j in range(HID_DIM // C)]
    b1 = b1_l.reshape(1, HID_DIM)
    b2 = b2_l.reshape(1, OUT_DIM)

    agg_lo, agg_hi, dp0, dp1 = _sc_agg_deg(x_lo, x_hi, src_p, dst_p, zblk, onesblk)
    r1 = _tc_mm([x], [Wt1r])
    hc = _tc_mm([agg_lo, agg_hi], Wt1l, bias=b1, degs=(dp0, dp1), resid=r1,
                relu=True, out_chunks=4)

    agg2 = _sc_agg4(hc[0], hc[1], hc[2], hc[3], src_p, dst_p, zblk)
    r2 = _tc_mm(list(hc), Wt2r)
    out = _tc_mm(list(agg2), Wt2l, bias=b2, degs=(dp0, dp1), resid=r2)
    return out


# speedup vs baseline: 1.0461x; 1.0461x over previous
import dataclasses
import functools

import jax
import jax.numpy as jnp
from jax import lax
from jax.experimental import pallas as pl
from jax.experimental.pallas import tpu as pltpu
from jax.experimental.pallas import tpu_sc as plsc

N = 10000
E = 160000
IN_DIM = 256
HID_DIM = 512
OUT_DIM = 512

NC = 2
NS = 16
N_PAD = 10240
ZROWS = N_PAD // NS
E_PAD = 163840
EPS = E_PAD // NS
BLK = 128
NBLK = EPS // BLK
C = 128

_f32 = jnp.float32

def _sc_mesh():
    return plsc.VectorSubcoreMesh(
        core_axis_name="c", subcore_axis_name="s", num_cores=NC, num_subcores=NS)


def _sc_params():
    cp = pltpu.CompilerParams()
    if "needs_layout_passes" in pltpu.CompilerParams.__dataclass_fields__:
        cp = dataclasses.replace(cp, needs_layout_passes=False)
    return cp


SLAB_B = 16
N_SLAB = NBLK // SLAB_B
DROWS = N_PAD // C


def _slab_pass(table_h, acc, src_slab, dst_slab, sts, gsems, ssems, dacc):
    ones_v = jnp.full((16,), 1.0, _f32)
    pltpu.async_copy(table_h.at[src_slab.at[0]], sts[0], gsems[0])

    @pl.loop(0, SLAB_B, step=2)
    def _(b):
        for cur in range(2):
            bb = b + cur
            nxt = 1 - cur
            pltpu.make_async_copy(table_h.at[src_slab.at[bb]], sts[cur], gsems[cur]).wait()
            pltpu.async_copy(sts[cur], acc.at[dst_slab.at[bb]], ssems[cur], add=True)
            if dacc is not None:
                for j in range(BLK // 16):
                    idx = dst_slab.at[bb][pl.ds(j * 16, 16)]
                    row = lax.shift_right_logical(idx, 7)
                    col = lax.bitwise_and(idx, 127)
                    plsc.addupdate_scatter(dacc, [row, col], ones_v)

            @pl.when(bb + 1 < SLAB_B)
            def _(bb=bb, cur=cur, nxt=nxt):
                @pl.when(bb >= 1)
                def _():
                    pltpu.make_async_copy(sts[nxt], acc.at[dst_slab.at[0]], ssems[nxt]).wait()

                pltpu.async_copy(table_h.at[src_slab.at[bb + 1]], sts[nxt], gsems[nxt])

    for i in range(2):
        pltpu.make_async_copy(sts[i], acc.at[dst_slab.at[0]], ssems[i]).wait()


def _agg_pass(table_h, src2_h, dst2_h, acc, src_slab, dst_slab, sts, gsems, ssems,
              s, dacc=None, deg_slabs=()):
    for h in range(N_SLAB):
        row0 = s * NBLK + h * SLAB_B
        pltpu.sync_copy(src2_h.at[pl.ds(row0, SLAB_B)], src_slab)
        pltpu.sync_copy(dst2_h.at[pl.ds(row0, SLAB_B)], dst_slab)
        _slab_pass(table_h, acc, src_slab, dst_slab, sts, gsems, ssems,
                   dacc if h in deg_slabs else None)


def _sc_agg_deg(x_lo, x_hi, src_p, dst_p, zblk):
    @functools.partial(
        pl.kernel,
        out_type=(jax.ShapeDtypeStruct((N_PAD, C), _f32),
                  jax.ShapeDtypeStruct((N_PAD, C), _f32),
                  jax.ShapeDtypeStruct((DROWS, C), _f32),
                  jax.ShapeDtypeStruct((DROWS, C), _f32)),
        mesh=_sc_mesh(),
        compiler_params=_sc_params(),
        scratch_types=[
            pltpu.VMEM_SHARED((N_PAD, C), _f32),
            pltpu.VMEM((SLAB_B, BLK), jnp.int32),
            pltpu.VMEM((SLAB_B, BLK), jnp.int32),
            pltpu.VMEM((BLK, C), _f32),
            pltpu.VMEM((BLK, C), _f32),
            pltpu.VMEM((DROWS, C), _f32),
            pltpu.SemaphoreType.DMA,
            pltpu.SemaphoreType.DMA,
            pltpu.SemaphoreType.DMA,
            pltpu.SemaphoreType.DMA,
        ],
    )
    def k(xlo_h, xhi_h, src2_h, dst2_h, zb_h,
          alo_h, ahi_h, dg0_h, dg1_h,
          acc, src_slab, dst_slab, st0, st1, dacc, g0, g1, s0, s1):
        c = lax.axis_index("c")
        s = lax.axis_index("s")
        rows = pl.ds(s * ZROWS, ZROWS)
        pltpu.sync_copy(zb_h, acc.at[rows])
        pltpu.sync_copy(zb_h.at[pl.ds(0, DROWS)], dacc)
        plsc.subcore_barrier()

        @pl.when(c == 0)
        def _():
            _agg_pass(xlo_h, src2_h, dst2_h, acc, src_slab, dst_slab, (st0, st1),
                      (g0, g1), (s0, s1), s, dacc, deg_slabs=(0, 1))

        @pl.when(c == 1)
        def _():
            _agg_pass(xhi_h, src2_h, dst2_h, acc, src_slab, dst_slab, (st0, st1),
                      (g0, g1), (s0, s1), s, dacc, deg_slabs=(2, 3, 4))

        plsc.subcore_barrier()

        @pl.when(c == 0)
        def _():
            pltpu.sync_copy(acc.at[rows], alo_h.at[rows])

        @pl.when(c == 1)
        def _():
            pltpu.sync_copy(acc.at[rows], ahi_h.at[rows])

        plsc.subcore_barrier()
        pltpu.sync_copy(dacc, acc.at[pl.ds(s * DROWS, DROWS)])
        plsc.subcore_barrier()

        @pl.when(s < DROWS // 8)
        def _():
            for r in range(8):
                for j in range(C // 16):
                    st0[r, pl.ds(j * 16, 16)] = jnp.zeros((16,), _f32)

            @pl.loop(0, NS)
            def _(t):
                pltpu.sync_copy(acc.at[pl.ds(t * DROWS + s * 8, 8)], st1.at[pl.ds(0, 8)])
                for r in range(8):
                    for j in range(C // 16):
                        sl = pl.ds(j * 16, 16)
                        st0[r, sl] = st0[r, sl] + st1[r, sl]

            @pl.when(c == 0)
            def _():
                pltpu.sync_copy(st0.at[pl.ds(0, 8)], dg0_h.at[pl.ds(s * 8, 8)])

            @pl.when(c == 1)
            def _():
                pltpu.sync_copy(st0.at[pl.ds(0, 8)], dg1_h.at[pl.ds(s * 8, 8)])

    return k(x_lo, x_hi, src_p, dst_p, zblk)


def _sc_agg4(h0, h1, h2, h3, src_p, dst_p, zblk):
    @functools.partial(
        pl.kernel,
        out_type=tuple(jax.ShapeDtypeStruct((N_PAD, C), _f32) for _ in range(4)),
        mesh=_sc_mesh(),
        scratch_types=[
            pltpu.VMEM_SHARED((N_PAD, C), _f32),
            pltpu.VMEM((SLAB_B, BLK), jnp.int32),
            pltpu.VMEM((SLAB_B, BLK), jnp.int32),
            pltpu.VMEM((BLK, C), _f32),
            pltpu.VMEM((BLK, C), _f32),
            pltpu.SemaphoreType.DMA,
            pltpu.SemaphoreType.DMA,
            pltpu.SemaphoreType.DMA,
            pltpu.SemaphoreType.DMA,
        ],
    )
    def k(h0_h, h1_h, h2_h, h3_h, src2_h, dst2_h, zb_h,
          o0_h, o1_h, o2_h, o3_h, acc, src_slab, dst_slab, st0, st1, g0, g1, s0, s1):
        c = lax.axis_index("c")
        s = lax.axis_index("s")
        rows = pl.ds(s * ZROWS, ZROWS)
        tables = (h0_h, h1_h, h2_h, h3_h)
        outs = (o0_h, o1_h, o2_h, o3_h)
        for p in range(2):
            pltpu.sync_copy(zb_h, acc.at[rows])
            plsc.subcore_barrier()
            for cc in range(NC):
                @pl.when(c == cc)
                def _(cc=cc, p=p):
                    _agg_pass(tables[2 * cc + p], src2_h, dst2_h, acc,
                              src_slab, dst_slab, (st0, st1), (g0, g1), (s0, s1), s)
            plsc.subcore_barrier()
            for cc in range(NC):
                @pl.when(c == cc)
                def _(cc=cc, p=p):
                    pltpu.sync_copy(acc.at[rows], outs[2 * cc + p].at[rows])
            if p == 0:
                plsc.subcore_barrier()

    return k(h0, h1, h2, h3, src_p, dst_p, zblk)


def _tc_mm(xs, ws, bias=None, degs=None, resid=None, relu=False, out_chunks=1):
    BN = 1000
    M = ws[0].shape[1]
    nx = len(xs)
    args = list(xs) + list(ws)
    in_specs = [pl.BlockSpec((BN, x.shape[1]), lambda i: (i, 0)) for x in xs]
    in_specs += [pl.BlockSpec(w.shape, lambda i: (0, 0)) for w in ws]
    have_bias, have_deg, have_resid = (bias is not None), (degs is not None), (resid is not None)
    if have_bias:
        in_specs.append(pl.BlockSpec((1, M), lambda i: (0, 0)))
        args.append(bias)
    if have_deg:
        for dp in degs:
            in_specs.append(pl.BlockSpec((BN, 1), lambda i: (i, 0)))
            args.append(dp)
    if have_resid:
        in_specs.append(pl.BlockSpec((BN, M), lambda i: (i, 0)))
        args.append(resid)
    if out_chunks == 1:
        out_shape = jax.ShapeDtypeStruct((N, M), _f32)
        out_specs = pl.BlockSpec((BN, M), lambda i: (i, 0))
    else:
        Mc = M // out_chunks
        out_shape = tuple(jax.ShapeDtypeStruct((N, Mc), _f32) for _ in range(out_chunks))
        out_specs = tuple(pl.BlockSpec((BN, Mc), lambda i: (i, 0)) for _ in range(out_chunks))

    def body(*refs):
        xrs = refs[:nx]
        wrs = refs[nx:2 * nx]
        pos = 2 * nx
        acc = jnp.dot(xrs[0][...], wrs[0][...], preferred_element_type=_f32)
        for j in range(1, nx):
            acc = acc + jnp.dot(xrs[j][...], wrs[j][...], preferred_element_type=_f32)
        if have_bias:
            b_ref = refs[pos]; pos += 1
        if have_deg:
            d = refs[pos][...] + refs[pos + 1][...]; pos += 2
            acc = acc / jnp.maximum(d, 1.0)
        if have_bias:
            acc = acc + b_ref[...]
        if have_resid:
            acc = acc + refs[pos][...]; pos += 1
        if relu:
            acc = jnp.maximum(acc, 0.0)
        outs = refs[pos:]
        if out_chunks == 1:
            outs[0][...] = acc
        else:
            for q in range(out_chunks):
                outs[q][...] = acc[:, q * (M // out_chunks):(q + 1) * (M // out_chunks)]

    return pl.pallas_call(
        body, grid=(N // BN,), in_specs=in_specs,
        out_specs=out_specs, out_shape=out_shape)(*args)


def kernel(x, edge_index, W1_l, b1_l, W1_r, W2_l, b2_l, W2_r):
    src = edge_index[0].astype(jnp.int32)
    dst = edge_index[1].astype(jnp.int32)
    pad = E_PAD - E
    src_p = jnp.concatenate([src, jnp.zeros((pad,), jnp.int32)]).reshape(E_PAD // BLK, BLK)
    dst_p = jnp.concatenate([dst, jnp.full((pad,), N, jnp.int32)]).reshape(E_PAD // BLK, BLK)
    x_lo = x[:, :C]
    x_hi = x[:, C:]
    zblk = jnp.zeros((ZROWS, C), _f32)
    Wt1l = [W1_l[:, j * C:(j + 1) * C].T for j in range(IN_DIM // C)]
    Wt1r = W1_r.T
    Wt2l = [W2_l[:, j * C:(j + 1) * C].T for j in range(HID_DIM // C)]
    Wt2r = [W2_r[:, j * C:(j + 1) * C].T for j in range(HID_DIM // C)]
    b1 = b1_l.reshape(1, HID_DIM)
    b2 = b2_l.reshape(1, OUT_DIM)

    agg_lo, agg_hi, dg0, dg1 = _sc_agg_deg(x_lo, x_hi, src_p, dst_p, zblk)
    dp0 = dg0.reshape(N_PAD, 1)
    dp1 = dg1.reshape(N_PAD, 1)
    r1 = _tc_mm([x], [Wt1r])
    hc = _tc_mm([agg_lo, agg_hi], Wt1l, bias=b1, degs=(dp0, dp1), resid=r1,
                relu=True, out_chunks=4)

    agg2 = _sc_agg4(hc[0], hc[1], hc[2], hc[3], src_p, dst_p, zblk)
    r2 = _tc_mm(list(hc), Wt2r)
    out = _tc_mm(list(agg2), Wt2l, bias=b2, degs=(dp0, dp1), resid=r2)
    return out
